# baseline (speedup 1.0000x reference)
import functools

import jax
import jax.numpy as jnp
from jax.experimental import pallas as pl
from jax.experimental.pallas import tpu as pltpu

HIDDEN = 2048
H = 16
HKV = 4
G = H // HKV
D = 64
BS = 64
SBLK = 16
WIN = 512
THETA = 10000.0
NEG = -1e9

RB = 256
TQ = 256


def _rope2d(x, cosb, sinb):
    j = jax.lax.broadcasted_iota(jnp.int32, x.shape, 1) % 64
    lo = jnp.roll(x, -32, axis=1)
    hi = jnp.roll(x, 32, axis=1)
    partner = jnp.where(j < 32, lo, hi)
    return x * cosb + partner * sinb


def _k1_body(hs_ref, wq_ref, wk_ref, wv_ref, wg_ref, cq_ref, sq_ref,
             ck_ref, sk_ref, q_ref, k_ref, v_ref, g_ref, kc_ref, vc_ref):
    hb = hs_ref[:]
    q = jnp.dot(hb, wq_ref[:], preferred_element_type=jnp.float32)
    q_ref[:] = _rope2d(q, cq_ref[:], sq_ref[:]) * (D ** -0.5)
    k = jnp.dot(hb, wk_ref[:], preferred_element_type=jnp.float32)
    kr = _rope2d(k, ck_ref[:], sk_ref[:])
    k_ref[:] = kr
    v = jnp.dot(hb, wv_ref[:], preferred_element_type=jnp.float32)
    v_ref[:] = v
    g_ref[:] = jax.nn.sigmoid(
        jnp.dot(hb, wg_ref[:], preferred_element_type=jnp.float32))
    nc = RB // BS
    ci = jax.lax.broadcasted_iota(jnp.int32, (nc, RB), 0)
    ri = jax.lax.broadcasted_iota(jnp.int32, (nc, RB), 1)
    P = jnp.where(ri // BS == ci, 1.0 / BS, 0.0).astype(jnp.float32)
    kc_ref[0] = jnp.dot(P, kr, preferred_element_type=jnp.float32)
    vc_ref[0] = jnp.dot(P, v, preferred_element_type=jnp.float32)


def _softmax_rows(s):
    m = jnp.max(s, axis=-1, keepdims=True)
    e = jnp.exp(s - m)
    return e / jnp.sum(e, axis=-1, keepdims=True)


def _k2_body(nqb, nc, sblk, q_ref, k_ref, v_ref, kc_ref, vc_ref, g_ref,
             wo_ref, e9_ref, swab_ref, crel_ref, out_ref, sb_ref):
    qi = pl.program_id(0)
    T = nc * BS
    CK = 256
    NCH = T // CK
    SW = min(768, T)

    @pl.when(pl.program_id(1) == 0)
    def _():
        out_ref[:] = jnp.zeros_like(out_ref)

    trow = qi * TQ + jax.lax.broadcasted_iota(jnp.int32, (TQ, 1), 0)
    c32 = jax.lax.broadcasted_iota(jnp.int32, (TQ, nc), 1)
    vis = trow >= (c32 + 1) * BS - 1
    selectable = c32 * BS <= trow
    cur = c32 == trow // BS

    kch = kc_ref[0]
    vch = vc_ref[0]

    nt = (((1,), (1,)), ((), ()))
    imp = jnp.zeros((TQ, nc), jnp.float32)
    o_cmp = []
    for g in range(G):
        qt = q_ref[0, g]
        sc = jax.lax.dot_general(qt, kch, nt,
                                 preferred_element_type=jnp.float32)
        p = _softmax_rows(jnp.where(vis, sc, NEG))
        p = jnp.where(vis, p, 0.0)
        imp = imp + p
        o_cmp.append(jnp.dot(p, vch, preferred_element_type=jnp.float32))
    impv = jnp.where(selectable, imp + jnp.where(cur, 1e9, 0.0), NEG)
    a = impv[:, None, :]
    b = impv[:, :, None]
    cpi = jax.lax.broadcasted_iota(jnp.int32, (1, nc, nc), 2)
    ci = jax.lax.broadcasted_iota(jnp.int32, (1, nc, nc), 1)
    gt = (a > b).astype(jnp.float32)
    eq = ((a == b) & (cpi < ci)).astype(jnp.float32)
    rank = jnp.sum(gt + eq, axis=2)
    sel = (rank < sblk).astype(jnp.float32)
    selbias = jnp.dot(sel - 1.0, e9_ref[:],
                      preferred_element_type=jnp.float32)
    for jj in range(NCH):
        sb_ref[jj] = selbias[:, jj * CK:(jj + 1) * CK]

    wstart = jnp.maximum(qi - 2, 0) * CK
    ksl = k_ref[0, pl.ds(wstart, SW)]
    vsl = v_ref[0, pl.ds(wstart, SW)]

    for g in range(G):
        qt = q_ref[0, g]
        sw = jax.lax.dot_general(qt, ksl, nt,
                                 preferred_element_type=jnp.float32)
        sw = sw + swab_ref[0]
        mw = jnp.max(sw, axis=-1, keepdims=True)
        ew = jnp.exp(sw - mw)
        dw = jnp.sum(ew, axis=-1, keepdims=True)
        o_swa = jnp.dot(ew, vsl, preferred_element_type=jnp.float32) / dw
        def chunk(j, carry, extra_bias=None):
            m, d, acc = carry
            kc_j = k_ref[0, pl.ds(j * CK, CK)]
            s = jax.lax.dot_general(qt, kc_j, nt,
                                    preferred_element_type=jnp.float32)
            s = s + sb_ref[j]
            if extra_bias is not None:
                s = s + extra_bias
            m_new = jnp.maximum(m, jnp.max(s, axis=-1, keepdims=True))
            alpha = jnp.exp(m - m_new)
            e = jnp.exp(s - m_new)
            d = d * alpha + jnp.sum(e, axis=-1, keepdims=True)
            acc = acc * alpha + jnp.dot(e, v_ref[0, pl.ds(j * CK, CK)],
                                        preferred_element_type=jnp.float32)
            return m_new, d, acc
        init = (jnp.full((TQ, 1), NEG, jnp.float32),
                jnp.zeros((TQ, 1), jnp.float32),
                jnp.zeros((TQ, D), jnp.float32))
        carry = jax.lax.fori_loop(0, qi, chunk, init)
        _, d, acc = chunk(qi, carry, extra_bias=crel_ref[:])
        o_slc = acc / d
        gb = g_ref[0, g]
        oh = (gb[:, 0:1] * o_cmp[g] + gb[:, 1:2] * o_slc
              + gb[:, 2:3] * o_swa)
        out_ref[:] += jnp.dot(oh, wo_ref[0, g],
                              preferred_element_type=jnp.float32)


def kernel(hidden_states, Wq, Wk, Wv, Wg, Wo):
    B, T, HID = hidden_states.shape
    hs = hidden_states.reshape(T, HID)
    nc = T // BS
    sblk = min(SBLK, nc)
    nrb = T // RB
    nqb = T // TQ

    inv = 1.0 / (THETA ** (jnp.arange(32, dtype=jnp.float32) / 32.0))
    fr = jnp.outer(jnp.arange(T, dtype=jnp.float32), inv)
    cosT, sinT = jnp.cos(fr), jnp.sin(fr)
    cq = jnp.tile(jnp.concatenate([cosT, cosT], axis=1), (1, H))
    sq = jnp.tile(jnp.concatenate([-sinT, sinT], axis=1), (1, H))
    ck = jnp.tile(jnp.concatenate([cosT, cosT], axis=1), (1, HKV))
    sk = jnp.tile(jnp.concatenate([-sinT, sinT], axis=1), (1, HKV))

    wqT, wkT, wvT, wgT = Wq.T, Wk.T, Wv.T, Wg.T

    full = lambda shape: pl.BlockSpec(shape, lambda i: tuple(0 for _ in shape))
    q2d, k2d, v2d, g2d, kc3, vc3 = pl.pallas_call(
        _k1_body,
        grid=(nrb,),
        in_specs=[
            pl.BlockSpec((RB, HID), lambda i: (i, 0)),
            full((HID, H * D)), full((HID, HKV * D)), full((HID, HKV * D)),
            full((HID, H * 3)),
            pl.BlockSpec((RB, H * D), lambda i: (i, 0)),
            pl.BlockSpec((RB, H * D), lambda i: (i, 0)),
            pl.BlockSpec((RB, HKV * D), lambda i: (i, 0)),
            pl.BlockSpec((RB, HKV * D), lambda i: (i, 0)),
        ],
        out_specs=[
            pl.BlockSpec((RB, H * D), lambda i: (i, 0)),
            pl.BlockSpec((RB, HKV * D), lambda i: (i, 0)),
            pl.BlockSpec((RB, HKV * D), lambda i: (i, 0)),
            pl.BlockSpec((RB, H * 3), lambda i: (i, 0)),
            pl.BlockSpec((1, RB // BS, HKV * D), lambda i: (i, 0, 0)),
            pl.BlockSpec((1, RB // BS, HKV * D), lambda i: (i, 0, 0)),
        ],
        out_shape=[
            jax.ShapeDtypeStruct((T, H * D), jnp.float32),
            jax.ShapeDtypeStruct((T, HKV * D), jnp.float32),
            jax.ShapeDtypeStruct((T, HKV * D), jnp.float32),
            jax.ShapeDtypeStruct((T, H * 3), jnp.float32),
            jax.ShapeDtypeStruct((nrb, RB // BS, HKV * D), jnp.float32),
            jax.ShapeDtypeStruct((nrb, RB // BS, HKV * D), jnp.float32),
        ],
    )(hs, wqT, wkT, wvT, wgT, cq, sq, ck, sk)

    q4 = q2d.reshape(T, HKV, G, D).transpose(1, 2, 0, 3)
    k4 = k2d.reshape(T, HKV, D).transpose(1, 0, 2)
    v4 = v2d.reshape(T, HKV, D).transpose(1, 0, 2)
    kc4 = kc3.reshape(nc, HKV, D).transpose(1, 0, 2)
    vc4 = vc3.reshape(nc, HKV, D).transpose(1, 0, 2)
    g4 = g2d.reshape(T, HKV, G, 3).transpose(1, 2, 0, 3)
    wo4 = Wo.T.reshape(HKV, G, D, HID)
    CK = 256
    NCH = T // CK
    SW = min(768, T)
    eci = jax.lax.broadcasted_iota(jnp.int32, (nc, T), 0)
    eti = jax.lax.broadcasted_iota(jnp.int32, (nc, T), 1)
    e9 = jnp.where(eti // BS == eci, 1e9, 0.0).astype(jnp.float32)
    qia = jnp.arange(nqb, dtype=jnp.int32)[:, None, None]
    ra = jnp.arange(TQ, dtype=jnp.int32)[None, :, None]
    ca = jnp.arange(SW, dtype=jnp.int32)[None, None, :]
    wstart = jnp.maximum(qia - 2, 0) * CK
    tt = qia * TQ + ra
    cc = wstart + ca
    swab = jnp.where((tt >= cc) & (tt - cc < WIN), 0.0, NEG).astype(jnp.float32)
    rr = jnp.arange(TQ, dtype=jnp.int32)[:, None]
    crel = jnp.where(rr >= jnp.arange(CK, dtype=jnp.int32)[None, :],
                     0.0, NEG).astype(jnp.float32)

    out = pl.pallas_call(
        functools.partial(_k2_body, nqb, nc, sblk),
        grid=(nqb, HKV),
        in_specs=[
            pl.BlockSpec((1, G, TQ, D), lambda i, h: (h, 0, i, 0)),
            pl.BlockSpec((1, T, D), lambda i, h: (h, 0, 0)),
            pl.BlockSpec((1, T, D), lambda i, h: (h, 0, 0)),
            pl.BlockSpec((1, nc, D), lambda i, h: (h, 0, 0)),
            pl.BlockSpec((1, nc, D), lambda i, h: (h, 0, 0)),
            pl.BlockSpec((1, G, TQ, 3), lambda i, h: (h, 0, i, 0)),
            pl.BlockSpec((1, G, D, HID), lambda i, h: (h, 0, 0, 0)),
            pl.BlockSpec((nc, T), lambda i, h: (0, 0)),
            pl.BlockSpec((1, TQ, SW), lambda i, h: (i, 0, 0)),
            pl.BlockSpec((TQ, CK), lambda i, h: (0, 0)),
        ],
        out_specs=pl.BlockSpec((TQ, HID), lambda i, h: (i, 0)),
        out_shape=jax.ShapeDtypeStruct((T, HID), jnp.float32),
        scratch_shapes=[pltpu.VMEM((NCH, TQ, CK), jnp.float32)],
    )(q4, k4, v4, kc4, vc4, g4, wo4, e9, swab, crel)

    return out.reshape(B, T, HID)


# speedup vs baseline: 1.2829x; 1.2829x over previous
import functools

import jax
import jax.numpy as jnp
from jax.experimental import pallas as pl
from jax.experimental.pallas import tpu as pltpu

HIDDEN = 2048
H = 16
HKV = 4
G = H // HKV
D = 64
BS = 64
SBLK = 16
WIN = 512
THETA = 10000.0
NEG = -1e9

RB = 256
TQ = 256


def _rope2d(x, cosb, sinb):
    j = jax.lax.broadcasted_iota(jnp.int32, x.shape, 1) % 64
    lo = jnp.roll(x, -32, axis=1)
    hi = jnp.roll(x, 32, axis=1)
    partner = jnp.where(j < 32, lo, hi)
    return x * cosb + partner * sinb


def _k1_body(hs_ref, wq_ref, wk_ref, wv_ref, wg_ref, cq_ref, sq_ref,
             ck_ref, sk_ref, q_ref, k_ref, v_ref, g_ref, kc_ref, vc_ref):
    hb = hs_ref[:]
    q = jnp.dot(hb, wq_ref[:], preferred_element_type=jnp.float32)
    q_ref[:] = _rope2d(q, cq_ref[:], sq_ref[:]) * (D ** -0.5)
    k = jnp.dot(hb, wk_ref[:], preferred_element_type=jnp.float32)
    kr = _rope2d(k, ck_ref[:], sk_ref[:])
    k_ref[:] = kr.astype(jnp.bfloat16)
    v = jnp.dot(hb, wv_ref[:], preferred_element_type=jnp.float32)
    v_ref[:] = v.astype(jnp.bfloat16)
    g_ref[:] = jax.nn.sigmoid(
        jnp.dot(hb, wg_ref[:], preferred_element_type=jnp.float32))
    nc = RB // BS
    ci = jax.lax.broadcasted_iota(jnp.int32, (nc, RB), 0)
    ri = jax.lax.broadcasted_iota(jnp.int32, (nc, RB), 1)
    P = jnp.where(ri // BS == ci, 1.0 / BS, 0.0).astype(jnp.float32)
    kc_ref[0] = jnp.dot(P, kr, preferred_element_type=jnp.float32)
    vc_ref[0] = jnp.dot(P, v, preferred_element_type=jnp.float32)


def _softmax_rows(s):
    m = jnp.max(s, axis=-1, keepdims=True)
    e = jnp.exp(s - m)
    return e / jnp.sum(e, axis=-1, keepdims=True)


def _k2_body(qi, kw, sw, nc, sblk, q_ref, k_ref, v_ref, kc_ref, vc_ref,
             g_ref, wo_ref, e9_ref, swab_ref, cpad_ref, out_ref):
    @pl.when(pl.program_id(0) == 0)
    def _():
        out_ref[:] = jnp.zeros_like(out_ref)

    trow = qi * TQ + jax.lax.broadcasted_iota(jnp.int32, (TQ, 1), 0)
    c32 = jax.lax.broadcasted_iota(jnp.int32, (TQ, nc), 1)
    vis = trow >= (c32 + 1) * BS - 1
    selectable = c32 * BS <= trow
    cur = c32 == trow // BS

    kch = kc_ref[0]
    vch = vc_ref[0]
    kh = k_ref[0]
    vh = v_ref[0]

    nt = (((1,), (1,)), ((), ()))
    imp = jnp.zeros((TQ, nc), jnp.float32)
    o_cmp = []
    for g in range(G):
        qt = q_ref[0, g]
        sc = jax.lax.dot_general(qt, kch, nt,
                                 preferred_element_type=jnp.float32)
        p = _softmax_rows(jnp.where(vis, sc, NEG))
        p = jnp.where(vis, p, 0.0)
        imp = imp + p
        o_cmp.append(jnp.dot(p, vch, preferred_element_type=jnp.float32))
    impv = jnp.where(selectable, imp + jnp.where(cur, 1e9, 0.0), NEG)
    a = impv[:, None, :]
    b = impv[:, :, None]
    cpi = jax.lax.broadcasted_iota(jnp.int32, (1, nc, nc), 2)
    ci = jax.lax.broadcasted_iota(jnp.int32, (1, nc, nc), 1)
    gt = (a > b).astype(jnp.float32)
    eq = ((a == b) & (cpi < ci)).astype(jnp.float32)
    rank = jnp.sum(gt + eq, axis=2)
    sel = (rank < sblk).astype(jnp.float32)
    bias = jnp.dot(sel - 1.0, e9_ref[:],
                   preferred_element_type=jnp.float32) + cpad_ref[:]

    for g in range(G):
        qt = q_ref[0, g].astype(jnp.bfloat16)
        s = jax.lax.dot_general(qt, kh, nt,
                                preferred_element_type=jnp.float32)
        ss = s + bias
        m = jnp.max(ss, axis=-1, keepdims=True)
        e = jnp.exp(ss - m).astype(jnp.bfloat16)
        d = jnp.sum(e.astype(jnp.float32), axis=-1, keepdims=True)
        o_slc = jnp.dot(e, vh, preferred_element_type=jnp.float32) / d
        sw_s = s[:, kw - sw:] + swab_ref[:]
        mw = jnp.max(sw_s, axis=-1, keepdims=True)
        ew = jnp.exp(sw_s - mw).astype(jnp.bfloat16)
        dw = jnp.sum(ew.astype(jnp.float32), axis=-1, keepdims=True)
        o_swa = jnp.dot(ew, vh[kw - sw:],
                        preferred_element_type=jnp.float32) / dw
        gb = g_ref[0, g]
        oh = (gb[:, 0:1] * o_cmp[g] + gb[:, 1:2] * o_slc
              + gb[:, 2:3] * o_swa)
        out_ref[:] += jnp.dot(oh.astype(jnp.bfloat16), wo_ref[0, g],
                              preferred_element_type=jnp.float32)


def kernel(hidden_states, Wq, Wk, Wv, Wg, Wo):
    B, T, HID = hidden_states.shape
    hs = hidden_states.reshape(T, HID)
    nc = T // BS
    sblk = min(SBLK, nc)
    nrb = T // RB
    nqb = T // TQ

    inv = 1.0 / (THETA ** (jnp.arange(32, dtype=jnp.float32) / 32.0))
    fr = jnp.outer(jnp.arange(T, dtype=jnp.float32), inv)
    cosT, sinT = jnp.cos(fr), jnp.sin(fr)
    cq = jnp.tile(jnp.concatenate([cosT, cosT], axis=1), (1, H))
    sq = jnp.tile(jnp.concatenate([-sinT, sinT], axis=1), (1, H))
    ck = jnp.tile(jnp.concatenate([cosT, cosT], axis=1), (1, HKV))
    sk = jnp.tile(jnp.concatenate([-sinT, sinT], axis=1), (1, HKV))

    wqT, wkT, wvT, wgT = Wq.T, Wk.T, Wv.T, Wg.T

    full = lambda shape: pl.BlockSpec(shape, lambda i: tuple(0 for _ in shape))
    q2d, k2d, v2d, g2d, kc3, vc3 = pl.pallas_call(
        _k1_body,
        grid=(nrb,),
        in_specs=[
            pl.BlockSpec((RB, HID), lambda i: (i, 0)),
            full((HID, H * D)), full((HID, HKV * D)), full((HID, HKV * D)),
            full((HID, H * 3)),
            pl.BlockSpec((RB, H * D), lambda i: (i, 0)),
            pl.BlockSpec((RB, H * D), lambda i: (i, 0)),
            pl.BlockSpec((RB, HKV * D), lambda i: (i, 0)),
            pl.BlockSpec((RB, HKV * D), lambda i: (i, 0)),
        ],
        out_specs=[
            pl.BlockSpec((RB, H * D), lambda i: (i, 0)),
            pl.BlockSpec((RB, HKV * D), lambda i: (i, 0)),
            pl.BlockSpec((RB, HKV * D), lambda i: (i, 0)),
            pl.BlockSpec((RB, H * 3), lambda i: (i, 0)),
            pl.BlockSpec((1, RB // BS, HKV * D), lambda i: (i, 0, 0)),
            pl.BlockSpec((1, RB // BS, HKV * D), lambda i: (i, 0, 0)),
        ],
        out_shape=[
            jax.ShapeDtypeStruct((T, H * D), jnp.float32),
            jax.ShapeDtypeStruct((T, HKV * D), jnp.bfloat16),
            jax.ShapeDtypeStruct((T, HKV * D), jnp.bfloat16),
            jax.ShapeDtypeStruct((T, H * 3), jnp.float32),
            jax.ShapeDtypeStruct((nrb, RB // BS, HKV * D), jnp.float32),
            jax.ShapeDtypeStruct((nrb, RB // BS, HKV * D), jnp.float32),
        ],
    )(hs, wqT, wkT, wvT, wgT, cq, sq, ck, sk)

    q4 = q2d.reshape(T, HKV, G, D).transpose(1, 2, 0, 3)
    k4 = k2d.reshape(T, HKV, D).transpose(1, 0, 2)
    v4 = v2d.reshape(T, HKV, D).transpose(1, 0, 2)
    kc4 = kc3.reshape(nc, HKV, D).transpose(1, 0, 2)
    vc4 = vc3.reshape(nc, HKV, D).transpose(1, 0, 2)
    g4 = g2d.reshape(T, HKV, G, 3).transpose(1, 2, 0, 3)
    wo4 = Wo.T.reshape(HKV, G, D, HID).astype(jnp.bfloat16)
    eci = jax.lax.broadcasted_iota(jnp.int32, (nc, T), 0)
    eti = jax.lax.broadcasted_iota(jnp.int32, (nc, T), 1)
    e9 = jnp.where(eti // BS == eci, 1e9, 0.0).astype(jnp.float32)

    rr = jnp.arange(TQ, dtype=jnp.int32)[:, None]
    crel = jnp.where(rr >= jnp.arange(TQ, dtype=jnp.int32)[None, :],
                     0.0, NEG).astype(jnp.float32)

    outs = []
    for qi in range(nqb):
        kw = (qi + 1) * TQ
        sw = min(WIN + TQ, kw)
        cpad = jnp.concatenate(
            [jnp.zeros((TQ, kw - TQ), jnp.float32), crel], axis=1)
        tt = qi * TQ + rr
        ccw = (kw - sw) + jnp.arange(sw, dtype=jnp.int32)[None, :]
        swab = jnp.where((tt >= ccw) & (tt - ccw < WIN), 0.0,
                         NEG).astype(jnp.float32)
        outs.append(pl.pallas_call(
            functools.partial(_k2_body, qi, kw, sw, nc, sblk),
            grid=(HKV,),
            in_specs=[
                pl.BlockSpec((1, G, TQ, D), lambda h, _qi=qi: (h, 0, _qi, 0)),
                pl.BlockSpec((1, kw, D), lambda h: (h, 0, 0)),
                pl.BlockSpec((1, kw, D), lambda h: (h, 0, 0)),
                pl.BlockSpec((1, nc, D), lambda h: (h, 0, 0)),
                pl.BlockSpec((1, nc, D), lambda h: (h, 0, 0)),
                pl.BlockSpec((1, G, TQ, 3), lambda h, _qi=qi: (h, 0, _qi, 0)),
                pl.BlockSpec((1, G, D, HID), lambda h: (h, 0, 0, 0)),
                pl.BlockSpec((nc, kw), lambda h: (0, 0)),
                pl.BlockSpec((TQ, sw), lambda h: (0, 0)),
                pl.BlockSpec((TQ, kw), lambda h: (0, 0)),
            ],
            out_specs=pl.BlockSpec((TQ, HID), lambda h: (0, 0)),
            out_shape=jax.ShapeDtypeStruct((TQ, HID), jnp.float32),
        )(q4, k4, v4, kc4, vc4, g4, wo4, e9[:, :kw], swab, cpad))

    out = jnp.concatenate(outs, axis=0)
    return out.reshape(B, T, HID)


# speedup vs baseline: 1.4928x; 1.1636x over previous
import functools

import jax
import jax.numpy as jnp
import numpy as np
from jax.experimental import pallas as pl
from jax.experimental.pallas import tpu as pltpu

HIDDEN = 2048
H = 16
HKV = 4
G = H // HKV
D = 64
BS = 64
SBLK = 16
WIN = 512
THETA = 10000.0
NEG = -1e9

RB = 256
TQ = 256


def _rope2d(x, cosb, sinb):
    j = jax.lax.broadcasted_iota(jnp.int32, x.shape, 1) % 64
    lo = jnp.roll(x, -32, axis=1)
    hi = jnp.roll(x, 32, axis=1)
    partner = jnp.where(j < 32, lo, hi)
    return x * cosb + partner * sinb


def _k1_body(hs_ref, wq_ref, wk_ref, wv_ref, wg_ref, cq_ref, sq_ref,
             ck_ref, sk_ref, q_ref, k_ref, v_ref, g_ref, kc_ref, vc_ref):
    nt = (((1,), (1,)), ((), ()))
    hb = hs_ref[:]
    q = jax.lax.dot_general(hb, wq_ref[:], nt,
                            preferred_element_type=jnp.float32)
    q_ref[:] = _rope2d(q, cq_ref[:], sq_ref[:]) * (D ** -0.5)
    k = jax.lax.dot_general(hb, wk_ref[:], nt,
                            preferred_element_type=jnp.float32)
    kr = _rope2d(k, ck_ref[:], sk_ref[:])
    k_ref[:] = kr.astype(jnp.bfloat16)
    v = jax.lax.dot_general(hb, wv_ref[:], nt,
                            preferred_element_type=jnp.float32)
    v_ref[:] = v.astype(jnp.bfloat16)
    g_ref[:] = jax.nn.sigmoid(
        jax.lax.dot_general(hb, wg_ref[:], nt,
                            preferred_element_type=jnp.float32))
    nc = RB // BS
    ci = jax.lax.broadcasted_iota(jnp.int32, (nc, RB), 0)
    ri = jax.lax.broadcasted_iota(jnp.int32, (nc, RB), 1)
    P = jnp.where(ri // BS == ci, 1.0 / BS, 0.0).astype(jnp.float32)
    kc_ref[0] = jnp.dot(P, kr, preferred_element_type=jnp.float32)
    vc_ref[0] = jnp.dot(P, v, preferred_element_type=jnp.float32)


def _softmax_rows(s):
    m = jnp.max(s, axis=-1, keepdims=True)
    e = jnp.exp(s - m)
    return e / jnp.sum(e, axis=-1, keepdims=True)


def _k2_body(qi, kw, sw, nc, sblk, q_ref, k_ref, v_ref, kc_ref, vc_ref,
             g_ref, wo_ref, e9_ref, swab_ref, cpad_ref, out_ref):
    @pl.when(pl.program_id(0) == 0)
    def _():
        out_ref[:] = jnp.zeros_like(out_ref)

    trow = qi * TQ + jax.lax.broadcasted_iota(jnp.int32, (TQ, 1), 0)
    c32 = jax.lax.broadcasted_iota(jnp.int32, (TQ, nc), 1)
    vis = trow >= (c32 + 1) * BS - 1
    selectable = c32 * BS <= trow
    cur = c32 == trow // BS

    kch = kc_ref[0]
    vch = vc_ref[0]
    kh = k_ref[0]
    vh = v_ref[0]

    nt = (((1,), (1,)), ((), ()))
    imp = jnp.zeros((TQ, nc), jnp.float32)
    o_cmp = []
    for g in range(G):
        qt = q_ref[0, g]
        sc = jax.lax.dot_general(qt, kch, nt,
                                 preferred_element_type=jnp.float32)
        p = _softmax_rows(jnp.where(vis, sc, NEG))
        p = jnp.where(vis, p, 0.0)
        imp = imp + p
        o_cmp.append(jnp.dot(p, vch, preferred_element_type=jnp.float32))
    impv = jnp.where(selectable, imp + jnp.where(cur, 1e9, 0.0), NEG)
    a = impv[:, None, :]
    b = impv[:, :, None]
    cpi = jax.lax.broadcasted_iota(jnp.int32, (1, nc, nc), 2)
    ci = jax.lax.broadcasted_iota(jnp.int32, (1, nc, nc), 1)
    gt = (a > b).astype(jnp.float32)
    eq = ((a == b) & (cpi < ci)).astype(jnp.float32)
    rank = jnp.sum(gt + eq, axis=2)
    sel = (rank < sblk).astype(jnp.float32)
    bias = jnp.dot(sel - 1.0, e9_ref[:],
                   preferred_element_type=jnp.float32) + cpad_ref[:]

    for g in range(G):
        qt = q_ref[0, g].astype(jnp.bfloat16)
        s = jax.lax.dot_general(qt, kh, nt,
                                preferred_element_type=jnp.float32)
        ss = s + bias
        m = jnp.max(ss, axis=-1, keepdims=True)
        e = jnp.exp(ss - m).astype(jnp.bfloat16)
        d = jnp.sum(e.astype(jnp.float32), axis=-1, keepdims=True)
        o_slc = jnp.dot(e, vh, preferred_element_type=jnp.float32) / d
        sw_s = s[:, kw - sw:] + swab_ref[:]
        mw = jnp.max(sw_s, axis=-1, keepdims=True)
        ew = jnp.exp(sw_s - mw).astype(jnp.bfloat16)
        dw = jnp.sum(ew.astype(jnp.float32), axis=-1, keepdims=True)
        o_swa = jnp.dot(ew, vh[kw - sw:],
                        preferred_element_type=jnp.float32) / dw
        gb = g_ref[0, g]
        oh = (gb[:, 0:1] * o_cmp[g] + gb[:, 1:2] * o_slc
              + gb[:, 2:3] * o_swa)
        out_ref[:] += jnp.dot(oh.astype(jnp.bfloat16), wo_ref[0, g],
                              preferred_element_type=jnp.float32)


def kernel(hidden_states, Wq, Wk, Wv, Wg, Wo):
    B, T, HID = hidden_states.shape
    hs = hidden_states.reshape(T, HID)
    nc = T // BS
    sblk = min(SBLK, nc)
    nrb = T // RB
    nqb = T // TQ

    inv = 1.0 / (THETA ** (np.arange(32, dtype=np.float32) / 32.0))
    fr = np.outer(np.arange(T, dtype=np.float32), inv)
    cosT = np.cos(fr).astype(np.float32)
    sinT = np.sin(fr).astype(np.float32)
    cq = np.tile(np.concatenate([cosT, cosT], axis=1), (1, H))
    sq = np.tile(np.concatenate([-sinT, sinT], axis=1), (1, H))
    ck = np.tile(np.concatenate([cosT, cosT], axis=1), (1, HKV))
    sk = np.tile(np.concatenate([-sinT, sinT], axis=1), (1, HKV))

    full = lambda shape: pl.BlockSpec(shape, lambda i: tuple(0 for _ in shape))
    q2d, k2d, v2d, g2d, kc3, vc3 = pl.pallas_call(
        _k1_body,
        grid=(nrb,),
        in_specs=[
            pl.BlockSpec((RB, HID), lambda i: (i, 0)),
            full((H * D, HID)), full((HKV * D, HID)), full((HKV * D, HID)),
            full((H * 3, HID)),
            pl.BlockSpec((RB, H * D), lambda i: (i, 0)),
            pl.BlockSpec((RB, H * D), lambda i: (i, 0)),
            pl.BlockSpec((RB, HKV * D), lambda i: (i, 0)),
            pl.BlockSpec((RB, HKV * D), lambda i: (i, 0)),
        ],
        out_specs=[
            pl.BlockSpec((RB, H * D), lambda i: (i, 0)),
            pl.BlockSpec((RB, HKV * D), lambda i: (i, 0)),
            pl.BlockSpec((RB, HKV * D), lambda i: (i, 0)),
            pl.BlockSpec((RB, H * 3), lambda i: (i, 0)),
            pl.BlockSpec((1, RB // BS, HKV * D), lambda i: (i, 0, 0)),
            pl.BlockSpec((1, RB // BS, HKV * D), lambda i: (i, 0, 0)),
        ],
        out_shape=[
            jax.ShapeDtypeStruct((T, H * D), jnp.float32),
            jax.ShapeDtypeStruct((T, HKV * D), jnp.bfloat16),
            jax.ShapeDtypeStruct((T, HKV * D), jnp.bfloat16),
            jax.ShapeDtypeStruct((T, H * 3), jnp.float32),
            jax.ShapeDtypeStruct((nrb, RB // BS, HKV * D), jnp.float32),
            jax.ShapeDtypeStruct((nrb, RB // BS, HKV * D), jnp.float32),
        ],
    )(hs, Wq, Wk, Wv, Wg, cq, sq, ck, sk)

    q4 = q2d.reshape(T, HKV, G, D).transpose(1, 2, 0, 3)
    k4 = k2d.reshape(T, HKV, D).transpose(1, 0, 2)
    v4 = v2d.reshape(T, HKV, D).transpose(1, 0, 2)
    kc4 = kc3.reshape(nc, HKV, D).transpose(1, 0, 2)
    vc4 = vc3.reshape(nc, HKV, D).transpose(1, 0, 2)
    g4 = g2d.reshape(T, HKV, G, 3).transpose(1, 2, 0, 3)
    wo4 = Wo.T.reshape(HKV, G, D, HID).astype(jnp.bfloat16)
    eci = np.arange(nc)[:, None]
    eti = np.arange(T)[None, :]
    e9 = np.where(eti // BS == eci, 1e9, 0.0).astype(np.float32)

    rr = np.arange(TQ)[:, None]
    crel = np.where(rr >= np.arange(TQ)[None, :], 0.0,
                    NEG).astype(np.float32)

    outs = []
    for qi in range(nqb):
        kw = (qi + 1) * TQ
        sw = min(WIN + TQ, kw)
        cpad = np.concatenate(
            [np.zeros((TQ, kw - TQ), np.float32), crel], axis=1)
        tt = qi * TQ + rr
        ccw = (kw - sw) + np.arange(sw)[None, :]
        swab = np.where((tt >= ccw) & (tt - ccw < WIN), 0.0,
                        NEG).astype(np.float32)
        outs.append(pl.pallas_call(
            functools.partial(_k2_body, qi, kw, sw, nc, sblk),
            grid=(HKV,),
            in_specs=[
                pl.BlockSpec((1, G, TQ, D), lambda h, _qi=qi: (h, 0, _qi, 0)),
                pl.BlockSpec((1, kw, D), lambda h: (h, 0, 0)),
                pl.BlockSpec((1, kw, D), lambda h: (h, 0, 0)),
                pl.BlockSpec((1, nc, D), lambda h: (h, 0, 0)),
                pl.BlockSpec((1, nc, D), lambda h: (h, 0, 0)),
                pl.BlockSpec((1, G, TQ, 3), lambda h, _qi=qi: (h, 0, _qi, 0)),
                pl.BlockSpec((1, G, D, HID), lambda h: (h, 0, 0, 0)),
                pl.BlockSpec((nc, kw), lambda h: (0, 0)),
                pl.BlockSpec((TQ, sw), lambda h: (0, 0)),
                pl.BlockSpec((TQ, kw), lambda h: (0, 0)),
            ],
            out_specs=pl.BlockSpec((TQ, HID), lambda h: (0, 0)),
            out_shape=jax.ShapeDtypeStruct((TQ, HID), jnp.float32),
        )(q4, k4, v4, kc4, vc4, g4, wo4, e9[:, :kw], swab, cpad))

    out = jnp.concatenate(outs, axis=0)
    return out.reshape(B, T, HID)


# speedup vs baseline: 1.5754x; 1.0553x over previous
import functools

import jax
import jax.numpy as jnp
import numpy as np
from jax.experimental import pallas as pl
from jax.experimental.pallas import tpu as pltpu

HIDDEN = 2048
H = 16
HKV = 4
G = H // HKV
D = 64
BS = 64
SBLK = 16
WIN = 512
THETA = 10000.0
NEG = -1e9

RB = 256
TQ = 256


def _rope2d(x, cosb, sinb):
    j = jax.lax.broadcasted_iota(jnp.int32, x.shape, 1) % 64
    lo = jnp.roll(x, -32, axis=1)
    hi = jnp.roll(x, 32, axis=1)
    partner = jnp.where(j < 32, lo, hi)
    return x * cosb + partner * sinb


def _k1_body(hs_ref, wq_ref, wk_ref, wv_ref, wg_ref, cq_ref, sq_ref,
             ck_ref, sk_ref, q_ref, k_ref, v_ref, g_ref, kc_ref, vc_ref):
    nt = (((1,), (1,)), ((), ()))
    hb = hs_ref[:]
    q = jax.lax.dot_general(hb, wq_ref[:], nt,
                            preferred_element_type=jnp.float32)
    q_ref[:] = _rope2d(q, cq_ref[:], sq_ref[:]) * (D ** -0.5)
    k = jax.lax.dot_general(hb, wk_ref[:], nt,
                            preferred_element_type=jnp.float32)
    kr = _rope2d(k, ck_ref[:], sk_ref[:])
    k_ref[:] = kr.astype(jnp.bfloat16)
    v = jax.lax.dot_general(hb, wv_ref[:], nt,
                            preferred_element_type=jnp.float32)
    v_ref[:] = v.astype(jnp.bfloat16)
    g_ref[:] = jax.nn.sigmoid(
        jax.lax.dot_general(hb, wg_ref[:], nt,
                            preferred_element_type=jnp.float32))
    nc = RB // BS
    ci = jax.lax.broadcasted_iota(jnp.int32, (nc, RB), 0)
    ri = jax.lax.broadcasted_iota(jnp.int32, (nc, RB), 1)
    P = jnp.where(ri // BS == ci, 1.0 / BS, 0.0).astype(jnp.float32)
    kc_ref[0] = jnp.dot(P, kr, preferred_element_type=jnp.float32)
    vc_ref[0] = jnp.dot(P, v, preferred_element_type=jnp.float32)


def _softmax_rows(s):
    m = jnp.max(s, axis=-1, keepdims=True)
    e = jnp.exp(s - m)
    return e / jnp.sum(e, axis=-1, keepdims=True)


def _k2_body(qi, kw, sw, nc, sblk, q_ref, k_ref, v_ref, kc_ref, vc_ref,
             g_ref, wo_ref, e9_ref, swab_ref, cpad_ref, out_ref):
    trow = qi * TQ + jax.lax.broadcasted_iota(jnp.int32, (TQ, 1), 0)
    c32 = jax.lax.broadcasted_iota(jnp.int32, (TQ, nc), 1)
    vis = trow >= (c32 + 1) * BS - 1
    selectable = c32 * BS <= trow
    cur = c32 == trow // BS
    cpi = jax.lax.broadcasted_iota(jnp.int32, (1, nc, nc), 2)
    ci = jax.lax.broadcasted_iota(jnp.int32, (1, nc, nc), 1)

    nt = (((1,), (1,)), ((), ()))
    out = jnp.zeros((TQ, HIDDEN), jnp.float32)
    for h in range(HKV):
        kch = kc_ref[:, h * D:(h + 1) * D]
        vch = vc_ref[:, h * D:(h + 1) * D]
        kh = k_ref[:, h * D:(h + 1) * D]
        vh = v_ref[:, h * D:(h + 1) * D]
        imp = jnp.zeros((TQ, nc), jnp.float32)
        o_cmp = []
        for g in range(G):
            hd = h * G + g
            qt = q_ref[:, hd * D:(hd + 1) * D]
            sc = jax.lax.dot_general(qt, kch, nt,
                                     preferred_element_type=jnp.float32)
            p = _softmax_rows(jnp.where(vis, sc, NEG))
            p = jnp.where(vis, p, 0.0)
            imp = imp + p
            o_cmp.append(jnp.dot(p, vch,
                                 preferred_element_type=jnp.float32))
        impv = jnp.where(selectable, imp + jnp.where(cur, 1e9, 0.0), NEG)
        a = impv[:, None, :]
        b = impv[:, :, None]
        gt = (a > b).astype(jnp.float32)
        eq = ((a == b) & (cpi < ci)).astype(jnp.float32)
        rank = jnp.sum(gt + eq, axis=2)
        sel = (rank < sblk).astype(jnp.float32)
        bias = jnp.dot(sel - 1.0, e9_ref[:],
                       preferred_element_type=jnp.float32) + cpad_ref[:]

        for g in range(G):
            hd = h * G + g
            qt = q_ref[:, hd * D:(hd + 1) * D].astype(jnp.bfloat16)
            s = jax.lax.dot_general(qt, kh, nt,
                                    preferred_element_type=jnp.float32)
            ss = s + bias
            m = jnp.max(ss, axis=-1, keepdims=True)
            e = jnp.exp(ss - m).astype(jnp.bfloat16)
            d = jnp.sum(e.astype(jnp.float32), axis=-1, keepdims=True)
            o_slc = jnp.dot(e, vh, preferred_element_type=jnp.float32) / d
            sw_s = s[:, kw - sw:] + swab_ref[:]
            mw = jnp.max(sw_s, axis=-1, keepdims=True)
            ew = jnp.exp(sw_s - mw).astype(jnp.bfloat16)
            dw = jnp.sum(ew.astype(jnp.float32), axis=-1, keepdims=True)
            o_swa = jnp.dot(ew, vh[kw - sw:],
                            preferred_element_type=jnp.float32) / dw
            gb = g_ref[:, 3 * hd:3 * hd + 3]
            oh = (gb[:, 0:1] * o_cmp[g] + gb[:, 1:2] * o_slc
                  + gb[:, 2:3] * o_swa)
            out = out + jax.lax.dot_general(
                oh.astype(jnp.bfloat16), wo_ref[:, hd * D:(hd + 1) * D],
                nt, preferred_element_type=jnp.float32)
    out_ref[:] = out


def kernel(hidden_states, Wq, Wk, Wv, Wg, Wo):
    B, T, HID = hidden_states.shape
    hs = hidden_states.reshape(T, HID)
    nc = T // BS
    sblk = min(SBLK, nc)
    nrb = T // RB
    nqb = T // TQ

    inv = 1.0 / (THETA ** (np.arange(32, dtype=np.float32) / 32.0))
    fr = np.outer(np.arange(T, dtype=np.float32), inv)
    cosT = np.cos(fr).astype(np.float32)
    sinT = np.sin(fr).astype(np.float32)
    cq = np.tile(np.concatenate([cosT, cosT], axis=1), (1, H))
    sq = np.tile(np.concatenate([-sinT, sinT], axis=1), (1, H))
    ck = np.tile(np.concatenate([cosT, cosT], axis=1), (1, HKV))
    sk = np.tile(np.concatenate([-sinT, sinT], axis=1), (1, HKV))

    full = lambda shape: pl.BlockSpec(shape, lambda i: tuple(0 for _ in shape))
    q2d, k2d, v2d, g2d, kc3, vc3 = pl.pallas_call(
        _k1_body,
        grid=(nrb,),
        in_specs=[
            pl.BlockSpec((RB, HID), lambda i: (i, 0)),
            full((H * D, HID)), full((HKV * D, HID)), full((HKV * D, HID)),
            full((H * 3, HID)),
            pl.BlockSpec((RB, H * D), lambda i: (i, 0)),
            pl.BlockSpec((RB, H * D), lambda i: (i, 0)),
            pl.BlockSpec((RB, HKV * D), lambda i: (i, 0)),
            pl.BlockSpec((RB, HKV * D), lambda i: (i, 0)),
        ],
        out_specs=[
            pl.BlockSpec((RB, H * D), lambda i: (i, 0)),
            pl.BlockSpec((RB, HKV * D), lambda i: (i, 0)),
            pl.BlockSpec((RB, HKV * D), lambda i: (i, 0)),
            pl.BlockSpec((RB, H * 3), lambda i: (i, 0)),
            pl.BlockSpec((1, RB // BS, HKV * D), lambda i: (i, 0, 0)),
            pl.BlockSpec((1, RB // BS, HKV * D), lambda i: (i, 0, 0)),
        ],
        out_shape=[
            jax.ShapeDtypeStruct((T, H * D), jnp.float32),
            jax.ShapeDtypeStruct((T, HKV * D), jnp.bfloat16),
            jax.ShapeDtypeStruct((T, HKV * D), jnp.bfloat16),
            jax.ShapeDtypeStruct((T, H * 3), jnp.float32),
            jax.ShapeDtypeStruct((nrb, RB // BS, HKV * D), jnp.float32),
            jax.ShapeDtypeStruct((nrb, RB // BS, HKV * D), jnp.float32),
        ],
    )(hs, Wq, Wk, Wv, Wg, cq, sq, ck, sk)

    kc2 = kc3.reshape(nc, HKV * D)
    vc2 = vc3.reshape(nc, HKV * D)
    wob = Wo.astype(jnp.bfloat16)
    eci = np.arange(nc)[:, None]
    eti = np.arange(T)[None, :]
    e9 = np.where(eti // BS == eci, 1e9, 0.0).astype(np.float32)

    rr = np.arange(TQ)[:, None]
    crel = np.where(rr >= np.arange(TQ)[None, :], 0.0,
                    NEG).astype(np.float32)

    outs = []
    for qi in range(nqb):
        kw = (qi + 1) * TQ
        sw = min(WIN + TQ, kw)
        cpad = np.concatenate(
            [np.zeros((TQ, kw - TQ), np.float32), crel], axis=1)
        tt = qi * TQ + rr
        ccw = (kw - sw) + np.arange(sw)[None, :]
        swab = np.where((tt >= ccw) & (tt - ccw < WIN), 0.0,
                        NEG).astype(np.float32)
        outs.append(pl.pallas_call(
            functools.partial(_k2_body, qi, kw, sw, nc, sblk),
            grid=(1,),
            in_specs=[
                pl.BlockSpec((TQ, H * D), lambda i, _qi=qi: (_qi, 0)),
                pl.BlockSpec((kw, HKV * D), lambda i: (0, 0)),
                pl.BlockSpec((kw, HKV * D), lambda i: (0, 0)),
                pl.BlockSpec((nc, HKV * D), lambda i: (0, 0)),
                pl.BlockSpec((nc, HKV * D), lambda i: (0, 0)),
                pl.BlockSpec((TQ, H * 3), lambda i, _qi=qi: (_qi, 0)),
                pl.BlockSpec((HID, H * D), lambda i: (0, 0)),
                pl.BlockSpec((nc, kw), lambda i: (0, 0)),
                pl.BlockSpec((TQ, sw), lambda i: (0, 0)),
                pl.BlockSpec((TQ, kw), lambda i: (0, 0)),
            ],
            out_specs=pl.BlockSpec((TQ, HID), lambda i: (0, 0)),
            out_shape=jax.ShapeDtypeStruct((TQ, HID), jnp.float32),
        )(q2d, k2d, v2d, kc2, vc2, g2d, wob, e9[:, :kw], swab, cpad))

    out = jnp.concatenate(outs, axis=0)
    return out.reshape(B, T, HID)


# speedup vs baseline: 1.6961x; 1.0766x over previous
import functools

import jax
import jax.numpy as jnp
import numpy as np
from jax.experimental import pallas as pl
from jax.experimental.pallas import tpu as pltpu

HIDDEN = 2048
H = 16
HKV = 4
G = H // HKV
D = 64
BS = 64
SBLK = 16
WIN = 512
THETA = 10000.0
NEG = -1e9

RB = 256
TQ = 256


def _rope2d(x, cosb, sinb):
    j = jax.lax.broadcasted_iota(jnp.int32, x.shape, 1) % 64
    lo = jnp.roll(x, -32, axis=1)
    hi = jnp.roll(x, 32, axis=1)
    partner = jnp.where(j < 32, lo, hi)
    return x * cosb + partner * sinb


def _k1_body(hs_ref, wq_ref, wk_ref, wv_ref, wg_ref, cq_ref, sq_ref,
             ck_ref, sk_ref, q_ref, k_ref, v_ref, g_ref, kc_ref, vc_ref):
    nt = (((1,), (1,)), ((), ()))
    hb = hs_ref[:]
    q = jax.lax.dot_general(hb, wq_ref[:], nt,
                            preferred_element_type=jnp.float32)
    q_ref[:] = _rope2d(q, cq_ref[:], sq_ref[:]) * (D ** -0.5)
    k = jax.lax.dot_general(hb, wk_ref[:], nt,
                            preferred_element_type=jnp.float32)
    kr = _rope2d(k, ck_ref[:], sk_ref[:])
    k_ref[:] = kr.astype(jnp.bfloat16)
    v = jax.lax.dot_general(hb, wv_ref[:], nt,
                            preferred_element_type=jnp.float32)
    v_ref[:] = v.astype(jnp.bfloat16)
    g_ref[:] = jax.nn.sigmoid(
        jax.lax.dot_general(hb, wg_ref[:], nt,
                            preferred_element_type=jnp.float32))
    nc = RB // BS
    ci = jax.lax.broadcasted_iota(jnp.int32, (nc, RB), 0)
    ri = jax.lax.broadcasted_iota(jnp.int32, (nc, RB), 1)
    P = jnp.where(ri // BS == ci, 1.0 / BS, 0.0).astype(jnp.float32)
    kc_ref[0] = jnp.dot(P, kr, preferred_element_type=jnp.float32)
    vc_ref[0] = jnp.dot(P, v, preferred_element_type=jnp.float32)


def _softmax_rows(s):
    m = jnp.max(s, axis=-1, keepdims=True)
    e = jnp.exp(s - m)
    return e / jnp.sum(e, axis=-1, keepdims=True)


def _attend_one(qi, r0, kw, sw, swpad, nc, sblk, q_ref, k_ref, v_ref,
                kc_ref, vc_ref, g_ref, wo_ref, e9_ref, swab_ref, crel_ref):
    trow = qi * TQ + jax.lax.broadcasted_iota(jnp.int32, (TQ, 1), 0)
    c32 = jax.lax.broadcasted_iota(jnp.int32, (TQ, nc), 1)
    vis = trow >= (c32 + 1) * BS - 1
    selectable = c32 * BS <= trow
    cur = c32 == trow // BS
    cpi = jax.lax.broadcasted_iota(jnp.int32, (1, nc, nc), 2)
    ci = jax.lax.broadcasted_iota(jnp.int32, (1, nc, nc), 1)

    nt = (((1,), (1,)), ((), ()))
    out = jnp.zeros((TQ, HIDDEN), jnp.float32)
    for h in range(HKV):
        kch = kc_ref[:, h * D:(h + 1) * D]
        vch = vc_ref[:, h * D:(h + 1) * D]
        kh = k_ref[:kw, h * D:(h + 1) * D]
        vh = v_ref[:kw, h * D:(h + 1) * D]
        imp = jnp.zeros((TQ, nc), jnp.float32)
        o_cmp = []
        for g in range(G):
            hd = h * G + g
            qt = q_ref[r0:r0 + TQ, hd * D:(hd + 1) * D]
            sc = jax.lax.dot_general(qt, kch, nt,
                                     preferred_element_type=jnp.float32)
            p = _softmax_rows(jnp.where(vis, sc, NEG))
            p = jnp.where(vis, p, 0.0)
            imp = imp + p
            o_cmp.append(jnp.dot(p, vch,
                                 preferred_element_type=jnp.float32))
        impv = jnp.where(selectable, imp + jnp.where(cur, 1e9, 0.0), NEG)
        a = impv[:, None, :]
        b = impv[:, :, None]
        gt = (a > b).astype(jnp.float32)
        eq = ((a == b) & (cpi < ci)).astype(jnp.float32)
        rank = jnp.sum(gt + eq, axis=2)
        sel = (rank < sblk).astype(jnp.float32)
        selbias = jnp.dot(sel - 1.0, e9_ref[:, :kw],
                          preferred_element_type=jnp.float32)
        bias_d = selbias[:, kw - TQ:] + crel_ref[:]

        for g in range(G):
            hd = h * G + g
            qt = q_ref[r0:r0 + TQ,
                       hd * D:(hd + 1) * D].astype(jnp.bfloat16)
            s = jax.lax.dot_general(qt, kh, nt,
                                    preferred_element_type=jnp.float32)
            sd = s[:, kw - TQ:] + bias_d
            md = jnp.max(sd, axis=-1, keepdims=True)
            if kw > TQ:
                sm = s[:, :kw - TQ] + selbias[:, :kw - TQ]
                m = jnp.maximum(jnp.max(sm, axis=-1, keepdims=True), md)
                em = jnp.exp(sm - m).astype(jnp.bfloat16)
                ed = jnp.exp(sd - m).astype(jnp.bfloat16)
                d = (jnp.sum(em.astype(jnp.float32), axis=-1, keepdims=True)
                     + jnp.sum(ed.astype(jnp.float32), axis=-1,
                               keepdims=True))
                o_slc = (jnp.dot(em, vh[:kw - TQ],
                                 preferred_element_type=jnp.float32)
                         + jnp.dot(ed, vh[kw - TQ:],
                                   preferred_element_type=jnp.float32)) / d
            else:
                ed = jnp.exp(sd - md).astype(jnp.bfloat16)
                d = jnp.sum(ed.astype(jnp.float32), axis=-1, keepdims=True)
                o_slc = jnp.dot(ed, vh,
                                preferred_element_type=jnp.float32) / d
            sw_s = s[:, kw - sw:] + swab_ref[:, swpad - sw:]
            mw = jnp.max(sw_s, axis=-1, keepdims=True)
            ew = jnp.exp(sw_s - mw).astype(jnp.bfloat16)
            dw = jnp.sum(ew.astype(jnp.float32), axis=-1, keepdims=True)
            o_swa = jnp.dot(ew, vh[kw - sw:],
                            preferred_element_type=jnp.float32) / dw
            gb = g_ref[r0:r0 + TQ, 3 * hd:3 * hd + 3]
            oh = (gb[:, 0:1] * o_cmp[g] + gb[:, 1:2] * o_slc
                  + gb[:, 2:3] * o_swa)
            out = out + jax.lax.dot_general(
                oh.astype(jnp.bfloat16), wo_ref[:, hd * D:(hd + 1) * D],
                nt, preferred_element_type=jnp.float32)
    return out


def _k2_multi(qlo, nqi, nc, sblk, swpad, q_ref, k_ref, v_ref, kc_ref,
              vc_ref, g_ref, wo_ref, e9_ref, swab_ref, crel_ref, out_ref):
    for ql in range(nqi):
        qi = qlo + ql
        kw = (qi + 1) * TQ
        sw = min(WIN + TQ, kw)
        out_ref[ql * TQ:(ql + 1) * TQ, :] = _attend_one(
            qi, ql * TQ, kw, sw, swpad, nc, sblk, q_ref, k_ref, v_ref,
            kc_ref, vc_ref, g_ref, wo_ref, e9_ref, swab_ref[ql], crel_ref)


def kernel(hidden_states, Wq, Wk, Wv, Wg, Wo):
    B, T, HID = hidden_states.shape
    hs = hidden_states.reshape(T, HID)
    nc = T // BS
    sblk = min(SBLK, nc)
    nrb = T // RB
    nqb = T // TQ

    inv = 1.0 / (THETA ** (np.arange(32, dtype=np.float32) / 32.0))
    fr = np.outer(np.arange(T, dtype=np.float32), inv)
    cosT = np.cos(fr).astype(np.float32)
    sinT = np.sin(fr).astype(np.float32)
    cq = np.tile(np.concatenate([cosT, cosT], axis=1), (1, H))
    sq = np.tile(np.concatenate([-sinT, sinT], axis=1), (1, H))
    ck = np.tile(np.concatenate([cosT, cosT], axis=1), (1, HKV))
    sk = np.tile(np.concatenate([-sinT, sinT], axis=1), (1, HKV))

    full = lambda shape: pl.BlockSpec(shape, lambda i: tuple(0 for _ in shape))
    q2d, k2d, v2d, g2d, kc3, vc3 = pl.pallas_call(
        _k1_body,
        grid=(nrb,),
        in_specs=[
            pl.BlockSpec((RB, HID), lambda i: (i, 0)),
            full((H * D, HID)), full((HKV * D, HID)), full((HKV * D, HID)),
            full((H * 3, HID)),
            pl.BlockSpec((RB, H * D), lambda i: (i, 0)),
            pl.BlockSpec((RB, H * D), lambda i: (i, 0)),
            pl.BlockSpec((RB, HKV * D), lambda i: (i, 0)),
            pl.BlockSpec((RB, HKV * D), lambda i: (i, 0)),
        ],
        out_specs=[
            pl.BlockSpec((RB, H * D), lambda i: (i, 0)),
            pl.BlockSpec((RB, HKV * D), lambda i: (i, 0)),
            pl.BlockSpec((RB, HKV * D), lambda i: (i, 0)),
            pl.BlockSpec((RB, H * 3), lambda i: (i, 0)),
            pl.BlockSpec((1, RB // BS, HKV * D), lambda i: (i, 0, 0)),
            pl.BlockSpec((1, RB // BS, HKV * D), lambda i: (i, 0, 0)),
        ],
        out_shape=[
            jax.ShapeDtypeStruct((T, H * D), jnp.float32),
            jax.ShapeDtypeStruct((T, HKV * D), jnp.bfloat16),
            jax.ShapeDtypeStruct((T, HKV * D), jnp.bfloat16),
            jax.ShapeDtypeStruct((T, H * 3), jnp.float32),
            jax.ShapeDtypeStruct((nrb, RB // BS, HKV * D), jnp.float32),
            jax.ShapeDtypeStruct((nrb, RB // BS, HKV * D), jnp.float32),
        ],
    )(hs, Wq, Wk, Wv, Wg, cq, sq, ck, sk)

    kc2 = kc3.reshape(nc, HKV * D)
    vc2 = vc3.reshape(nc, HKV * D)
    wob = Wo.astype(jnp.bfloat16)
    eci = np.arange(nc)[:, None]
    eti = np.arange(T)[None, :]
    e9 = np.where(eti // BS == eci, 1e9, 0.0).astype(np.float32)

    rr = np.arange(TQ)[:, None]
    crel = np.where(rr >= np.arange(TQ)[None, :], 0.0,
                    NEG).astype(np.float32)

    outs = []
    for qlo in range(0, nqb, 4):
        nqi = min(4, nqb - qlo)
        kwmax = (qlo + nqi) * TQ
        swpad = min(WIN + TQ, kwmax)
        swab = np.full((nqi, TQ, swpad), NEG, np.float32)
        for ql in range(nqi):
            qi = qlo + ql
            kw = (qi + 1) * TQ
            sw = min(WIN + TQ, kw)
            tt = qi * TQ + rr
            ccw = (kw - sw) + np.arange(sw)[None, :]
            swab[ql, :, swpad - sw:] = np.where(
                (tt >= ccw) & (tt - ccw < WIN), 0.0, NEG)
        outs.append(pl.pallas_call(
            functools.partial(_k2_multi, qlo, nqi, nc, sblk, swpad),
            grid=(1,),
            in_specs=[
                pl.BlockSpec((nqi * TQ, H * D),
                             lambda i, _b=qlo // 4: (_b, 0)),
                pl.BlockSpec((kwmax, HKV * D), lambda i: (0, 0)),
                pl.BlockSpec((kwmax, HKV * D), lambda i: (0, 0)),
                pl.BlockSpec((nc, HKV * D), lambda i: (0, 0)),
                pl.BlockSpec((nc, HKV * D), lambda i: (0, 0)),
                pl.BlockSpec((nqi * TQ, H * 3),
                             lambda i, _b=qlo // 4: (_b, 0)),
                pl.BlockSpec((HID, H * D), lambda i: (0, 0)),
                pl.BlockSpec((nc, kwmax), lambda i: (0, 0)),
                pl.BlockSpec((nqi, TQ, swpad), lambda i: (0, 0, 0)),
                pl.BlockSpec((TQ, TQ), lambda i: (0, 0)),
            ],
            out_specs=pl.BlockSpec((nqi * TQ, HID), lambda i: (0, 0)),
            out_shape=jax.ShapeDtypeStruct((nqi * TQ, HID), jnp.float32),
        )(q2d, k2d, v2d, kc2, vc2, g2d, wob, e9[:, :kwmax], swab, crel))

    out = jnp.concatenate(outs, axis=0) if len(outs) > 1 else outs[0]
    return out.reshape(B, T, HID)


# speedup vs baseline: 1.9150x; 1.1291x over previous
import functools

import jax
import jax.numpy as jnp
import numpy as np
from jax.experimental import pallas as pl
from jax.experimental.pallas import tpu as pltpu

HIDDEN = 2048
H = 16
HKV = 4
G = H // HKV
D = 64
BS = 64
SBLK = 16
WIN = 512
THETA = 10000.0
NEG = -1e9

RB = 256
TQ = 256


def _rope2d(x, cosb, sinb):
    j = jax.lax.broadcasted_iota(jnp.int32, x.shape, 1) % 64
    lo = jnp.roll(x, -32, axis=1)
    hi = jnp.roll(x, 32, axis=1)
    partner = jnp.where(j < 32, lo, hi)
    return x * cosb + partner * sinb


def _k1_body(hs_ref, wq_ref, wk_ref, wv_ref, wg_ref, cq_ref, sq_ref,
             ck_ref, sk_ref, q_ref, k_ref, v_ref, g_ref, kc_ref, vc_ref):
    nt = (((1,), (1,)), ((), ()))
    hb = hs_ref[:]
    q = jax.lax.dot_general(hb, wq_ref[:], nt,
                            preferred_element_type=jnp.float32)
    q_ref[:] = _rope2d(q, cq_ref[:], sq_ref[:]) * (D ** -0.5)
    k = jax.lax.dot_general(hb, wk_ref[:], nt,
                            preferred_element_type=jnp.float32)
    kr = _rope2d(k, ck_ref[:], sk_ref[:])
    k_ref[:] = kr.astype(jnp.bfloat16)
    v = jax.lax.dot_general(hb, wv_ref[:], nt,
                            preferred_element_type=jnp.float32)
    v_ref[:] = v.astype(jnp.bfloat16)
    g_ref[:] = jax.nn.sigmoid(
        jax.lax.dot_general(hb, wg_ref[:], nt,
                            preferred_element_type=jnp.float32))
    nc = RB // BS
    ci = jax.lax.broadcasted_iota(jnp.int32, (nc, RB), 0)
    ri = jax.lax.broadcasted_iota(jnp.int32, (nc, RB), 1)
    P = jnp.where(ri // BS == ci, 1.0 / BS, 0.0).astype(jnp.float32)
    kc_ref[0] = jnp.dot(P, kr, preferred_element_type=jnp.float32)
    vc_ref[0] = jnp.dot(P, v, preferred_element_type=jnp.float32)


def _softmax_rows(s):
    m = jnp.max(s, axis=-1, keepdims=True)
    e = jnp.exp(s - m)
    return e / jnp.sum(e, axis=-1, keepdims=True)


def _attend_one(qi, r0, kw, sw, swpad, nc, sblk, q_ref, k_ref, v_ref,
                kc_ref, vc_ref, g_ref, wo_ref, e9_ref, swab_ref, crel_ref):
    trow = qi * TQ + jax.lax.broadcasted_iota(jnp.int32, (TQ, 1), 0)
    c32 = jax.lax.broadcasted_iota(jnp.int32, (TQ, nc), 1)
    vis = trow >= (c32 + 1) * BS - 1
    selectable = c32 * BS <= trow
    cur = c32 == trow // BS
    cpi = jax.lax.broadcasted_iota(jnp.int32, (1, nc, nc), 2)
    ci = jax.lax.broadcasted_iota(jnp.int32, (1, nc, nc), 1)

    nt = (((1,), (1,)), ((), ()))
    ohs = []
    for h in range(HKV):
        kch = kc_ref[:, h * D:(h + 1) * D]
        vch = vc_ref[:, h * D:(h + 1) * D]
        kh = k_ref[:kw, h * D:(h + 1) * D]
        vh = v_ref[:kw, h * D:(h + 1) * D]
        imp = jnp.zeros((TQ, nc), jnp.float32)
        o_cmp = []
        for g in range(G):
            hd = h * G + g
            qt = q_ref[r0:r0 + TQ, hd * D:(hd + 1) * D]
            sc = jax.lax.dot_general(qt, kch, nt,
                                     preferred_element_type=jnp.float32)
            p = _softmax_rows(jnp.where(vis, sc, NEG))
            p = jnp.where(vis, p, 0.0)
            imp = imp + p
            o_cmp.append(jnp.dot(p, vch,
                                 preferred_element_type=jnp.float32))
        impv = jnp.where(selectable, imp + jnp.where(cur, 1e9, 0.0), NEG)
        a = impv[:, None, :]
        b = impv[:, :, None]
        gt = (a > b).astype(jnp.float32)
        eq = ((a == b) & (cpi < ci)).astype(jnp.float32)
        rank = jnp.sum(gt + eq, axis=2)
        sel = (rank < sblk).astype(jnp.float32)
        selbias = jnp.dot(sel - 1.0, e9_ref[:, :kw],
                          preferred_element_type=jnp.float32)
        bias_d = selbias[:, kw - TQ:] + crel_ref[:]

        for g in range(G):
            hd = h * G + g
            qt = q_ref[r0:r0 + TQ,
                       hd * D:(hd + 1) * D].astype(jnp.bfloat16)
            s = jax.lax.dot_general(qt, kh, nt,
                                    preferred_element_type=jnp.float32)
            sd = s[:, kw - TQ:] + bias_d
            md = jnp.max(sd, axis=-1, keepdims=True)
            if kw > TQ:
                sm = s[:, :kw - TQ] + selbias[:, :kw - TQ]
                m = jnp.maximum(jnp.max(sm, axis=-1, keepdims=True), md)
                emf = jnp.exp(sm - m)
                edf = jnp.exp(sd - m)
                d = (jnp.sum(emf, axis=-1, keepdims=True)
                     + jnp.sum(edf, axis=-1, keepdims=True))
                o_slc = (jnp.dot(emf.astype(jnp.bfloat16), vh[:kw - TQ],
                                 preferred_element_type=jnp.float32)
                         + jnp.dot(edf.astype(jnp.bfloat16), vh[kw - TQ:],
                                   preferred_element_type=jnp.float32)) / d
            else:
                edf = jnp.exp(sd - md)
                d = jnp.sum(edf, axis=-1, keepdims=True)
                o_slc = jnp.dot(edf.astype(jnp.bfloat16), vh,
                                preferred_element_type=jnp.float32) / d
            sw_s = s[:, kw - sw:] + swab_ref[:, swpad - sw:]
            mw = jnp.max(sw_s, axis=-1, keepdims=True)
            ewf = jnp.exp(sw_s - mw)
            dw = jnp.sum(ewf, axis=-1, keepdims=True)
            o_swa = jnp.dot(ewf.astype(jnp.bfloat16), vh[kw - sw:],
                            preferred_element_type=jnp.float32) / dw
            gb = g_ref[r0:r0 + TQ, 3 * hd:3 * hd + 3]
            oh = (gb[:, 0:1] * o_cmp[g] + gb[:, 1:2] * o_slc
                  + gb[:, 2:3] * o_swa)
            ohs.append(oh.astype(jnp.bfloat16))
    return jax.lax.dot_general(jnp.concatenate(ohs, axis=1), wo_ref[:],
                               nt, preferred_element_type=jnp.float32)


def _k2_multi(qlo, nqi, nc, sblk, swpad, q_ref, k_ref, v_ref, kc_ref,
              vc_ref, g_ref, wo_ref, e9_ref, swab_ref, crel_ref, out_ref):
    for ql in range(nqi):
        qi = qlo + ql
        kw = (qi + 1) * TQ
        sw = min(WIN + TQ, kw)
        out_ref[ql * TQ:(ql + 1) * TQ, :] = _attend_one(
            qi, ql * TQ, kw, sw, swpad, nc, sblk, q_ref, k_ref, v_ref,
            kc_ref, vc_ref, g_ref, wo_ref, e9_ref, swab_ref[ql], crel_ref)


def kernel(hidden_states, Wq, Wk, Wv, Wg, Wo):
    B, T, HID = hidden_states.shape
    hs = hidden_states.reshape(T, HID)
    nc = T // BS
    sblk = min(SBLK, nc)
    nrb = T // RB
    nqb = T // TQ

    inv = 1.0 / (THETA ** (np.arange(32, dtype=np.float32) / 32.0))
    fr = np.outer(np.arange(T, dtype=np.float32), inv)
    cosT = np.cos(fr).astype(np.float32)
    sinT = np.sin(fr).astype(np.float32)
    cq = np.tile(np.concatenate([cosT, cosT], axis=1), (1, H))
    sq = np.tile(np.concatenate([-sinT, sinT], axis=1), (1, H))
    ck = np.tile(np.concatenate([cosT, cosT], axis=1), (1, HKV))
    sk = np.tile(np.concatenate([-sinT, sinT], axis=1), (1, HKV))

    full = lambda shape: pl.BlockSpec(shape, lambda i: tuple(0 for _ in shape))
    q2d, k2d, v2d, g2d, kc3, vc3 = pl.pallas_call(
        _k1_body,
        grid=(nrb,),
        in_specs=[
            pl.BlockSpec((RB, HID), lambda i: (i, 0)),
            full((H * D, HID)), full((HKV * D, HID)), full((HKV * D, HID)),
            full((H * 3, HID)),
            pl.BlockSpec((RB, H * D), lambda i: (i, 0)),
            pl.BlockSpec((RB, H * D), lambda i: (i, 0)),
            pl.BlockSpec((RB, HKV * D), lambda i: (i, 0)),
            pl.BlockSpec((RB, HKV * D), lambda i: (i, 0)),
        ],
        out_specs=[
            pl.BlockSpec((RB, H * D), lambda i: (i, 0)),
            pl.BlockSpec((RB, HKV * D), lambda i: (i, 0)),
            pl.BlockSpec((RB, HKV * D), lambda i: (i, 0)),
            pl.BlockSpec((RB, H * 3), lambda i: (i, 0)),
            pl.BlockSpec((1, RB // BS, HKV * D), lambda i: (i, 0, 0)),
            pl.BlockSpec((1, RB // BS, HKV * D), lambda i: (i, 0, 0)),
        ],
        out_shape=[
            jax.ShapeDtypeStruct((T, H * D), jnp.float32),
            jax.ShapeDtypeStruct((T, HKV * D), jnp.bfloat16),
            jax.ShapeDtypeStruct((T, HKV * D), jnp.bfloat16),
            jax.ShapeDtypeStruct((T, H * 3), jnp.float32),
            jax.ShapeDtypeStruct((nrb, RB // BS, HKV * D), jnp.float32),
            jax.ShapeDtypeStruct((nrb, RB // BS, HKV * D), jnp.float32),
        ],
    )(hs, Wq, Wk, Wv, Wg, cq, sq, ck, sk)

    kc2 = kc3.reshape(nc, HKV * D)
    vc2 = vc3.reshape(nc, HKV * D)
    wob = Wo.astype(jnp.bfloat16)
    eci = np.arange(nc)[:, None]
    eti = np.arange(T)[None, :]
    e9 = np.where(eti // BS == eci, 1e9, 0.0).astype(np.float32)

    rr = np.arange(TQ)[:, None]
    crel = np.where(rr >= np.arange(TQ)[None, :], 0.0,
                    NEG).astype(np.float32)

    outs = []
    for qlo in range(0, nqb, 4):
        nqi = min(4, nqb - qlo)
        kwmax = (qlo + nqi) * TQ
        swpad = min(WIN + TQ, kwmax)
        swab = np.full((nqi, TQ, swpad), NEG, np.float32)
        for ql in range(nqi):
            qi = qlo + ql
            kw = (qi + 1) * TQ
            sw = min(WIN + TQ, kw)
            tt = qi * TQ + rr
            ccw = (kw - sw) + np.arange(sw)[None, :]
            swab[ql, :, swpad - sw:] = np.where(
                (tt >= ccw) & (tt - ccw < WIN), 0.0, NEG)
        outs.append(pl.pallas_call(
            functools.partial(_k2_multi, qlo, nqi, nc, sblk, swpad),
            grid=(1,),
            in_specs=[
                pl.BlockSpec((nqi * TQ, H * D),
                             lambda i, _b=qlo // 4: (_b, 0)),
                pl.BlockSpec((kwmax, HKV * D), lambda i: (0, 0)),
                pl.BlockSpec((kwmax, HKV * D), lambda i: (0, 0)),
                pl.BlockSpec((nc, HKV * D), lambda i: (0, 0)),
                pl.BlockSpec((nc, HKV * D), lambda i: (0, 0)),
                pl.BlockSpec((nqi * TQ, H * 3),
                             lambda i, _b=qlo // 4: (_b, 0)),
                pl.BlockSpec((HID, H * D), lambda i: (0, 0)),
                pl.BlockSpec((nc, kwmax), lambda i: (0, 0)),
                pl.BlockSpec((nqi, TQ, swpad), lambda i: (0, 0, 0)),
                pl.BlockSpec((TQ, TQ), lambda i: (0, 0)),
            ],
            out_specs=pl.BlockSpec((nqi * TQ, HID), lambda i: (0, 0)),
            out_shape=jax.ShapeDtypeStruct((nqi * TQ, HID), jnp.float32),
        )(q2d, k2d, v2d, kc2, vc2, g2d, wob, e9[:, :kwmax], swab, crel))

    out = jnp.concatenate(outs, axis=0) if len(outs) > 1 else outs[0]
    return out.reshape(B, T, HID)
